# Initial kernel scaffold; baseline (speedup 1.0000x reference)
import jax
import jax.numpy as jnp
from jax.experimental import pallas as pl


def kernel(x, w_down, bn1_g, bn1_b, gcn_w, gcn_b, w_up, bn2_g, bn2_b, batch, span):
    raise NotImplementedError("write your pallas kernel here")


# speedup vs baseline: 10.7250x; 10.7250x over previous
import functools
from typing import Any

import jax
import jax.numpy as jnp
from jax import lax
from jax.experimental import pallas as pl
from jax.experimental.pallas import tpu as pltpu

B = 4
V = 8
C = 256
N = 196
T1 = V - 1
K = 8
NODES = V * N
E = 2 * T1 * K
CNT = B * V * N


def _conv1_body(xm_ref, xc_ref, xp_ref, w_ref, y_ref, s1_ref, s2_ref):
    t = pl.program_id(1)
    b = pl.program_id(0)
    first = jnp.logical_and(b == 0, t == 0)

    @pl.when(first)
    def _():
        s1_ref[...] = jnp.zeros_like(s1_ref)
        s2_ref[...] = jnp.zeros_like(s2_ref)

    xm = xm_ref[0]
    xc = xc_ref[0]
    xp = xp_ref[0]
    w = w_ref[...]
    dn = (((0,), (1,)), ((), ()))
    y = lax.dot_general(xc, w[1], dn, preferred_element_type=jnp.float32)
    ym = lax.dot_general(xm, w[0], dn, preferred_element_type=jnp.float32)
    yp = lax.dot_general(xp, w[2], dn, preferred_element_type=jnp.float32)
    mm = jnp.where(t > 0, 1.0, 0.0).astype(jnp.float32)
    mp = jnp.where(t < V - 1, 1.0, 0.0).astype(jnp.float32)
    y = y + mm * ym + mp * yp
    y_ref[0, 0] = y
    s1_ref[...] += jnp.sum(y, axis=0, keepdims=True)
    s2_ref[...] += jnp.sum(y * y, axis=0, keepdims=True)


def _conv1(x3, w3):
    grid = (B, V)
    return pl.pallas_call(
        _conv1_body,
        grid=grid,
        in_specs=[
            pl.BlockSpec((1, C, N), lambda b, t: (b * V + jnp.maximum(t - 1, 0), 0, 0)),
            pl.BlockSpec((1, C, N), lambda b, t: (b * V + t, 0, 0)),
            pl.BlockSpec((1, C, N), lambda b, t: (b * V + jnp.minimum(t + 1, V - 1), 0, 0)),
            pl.BlockSpec((3, C, C), lambda b, t: (0, 0, 0)),
        ],
        out_specs=[
            pl.BlockSpec((1, 1, N, C), lambda b, t: (b, t, 0, 0)),
            pl.BlockSpec((1, C), lambda b, t: (0, 0)),
            pl.BlockSpec((1, C), lambda b, t: (0, 0)),
        ],
        out_shape=[
            jax.ShapeDtypeStruct((B, V, N, C), jnp.float32),
            jax.ShapeDtypeStruct((1, C), jnp.float32),
            jax.ShapeDtypeStruct((1, C), jnp.float32),
        ],
    )(x3, x3, x3, w3)


def _topk_body(ya_ref, yb_ref, sc_ref, sh_ref, idx_ref):
    s = sc_ref[...]
    sh = sh_ref[...]
    xa = ya_ref[0, 0] * s + sh
    xb = yb_ref[0, 0] * s + sh
    a2 = jnp.sum(xa * xa, axis=1, keepdims=True)
    b2 = jnp.sum(xb * xb, axis=1, keepdims=True)
    g = lax.dot_general(xa, xb, (((1,), (1,)), ((), ())),
                        preferred_element_type=jnp.float32)
    d2 = jnp.clip(a2 + b2.T - 2.0 * g, 0.0, None)
    sim = -jnp.sqrt(d2)
    nrm = jnp.sqrt(jnp.sum(sim * sim))
    simn = sim / jnp.maximum(nrm, 1e-12)
    iota = lax.broadcasted_iota(jnp.int32, (N, N), 0) * N + \
        lax.broadcasted_iota(jnp.int32, (N, N), 1)
    big = jnp.int32(2**30)
    kiota = lax.broadcasted_iota(jnp.int32, (1, 1, K), 2)
    acc = jnp.zeros((1, 1, K), jnp.int32)
    for k in range(K):
        m = jnp.max(simn)
        fidx = jnp.min(jnp.where(simn == m, iota, big))
        acc = jnp.where(kiota == k, fidx, acc)
        simn = jnp.where(iota == fidx, -jnp.float32(jnp.inf), simn)
    idx_ref[...] = acc


def _topk(y, scale, shift):
    grid = (B * T1,)
    return pl.pallas_call(
        _topk_body,
        grid=grid,
        in_specs=[
            pl.BlockSpec((1, 1, N, C), lambda i: (i // T1, i % T1, 0, 0)),
            pl.BlockSpec((1, 1, N, C), lambda i: (i // T1, i % T1 + 1, 0, 0)),
            pl.BlockSpec((1, C), lambda i: (0, 0)),
            pl.BlockSpec((1, C), lambda i: (0, 0)),
        ],
        out_specs=pl.BlockSpec((1, 1, K), lambda i: (i, 0, 0)),
        out_shape=jax.ShapeDtypeStruct((B * T1, 1, K), jnp.int32),
    )(y, y, scale, shift)


def _gcn_body(y_ref, e_ref, sc_ref, sh_ref, gw_ref, gb_ref, wu_ref,
              z_ref, s1_ref, s2_ref):
    b = pl.program_id(0)

    @pl.when(b == 0)
    def _():
        s1_ref[...] = jnp.zeros_like(s1_ref)
        s2_ref[...] = jnp.zeros_like(s2_ref)

    xn = y_ref[0] * sc_ref[...] + sh_ref[...]
    h = lax.dot_general(xn, gw_ref[...], (((1,), (1,)), ((), ())),
                        preferred_element_type=jnp.float32)

    src = e_ref[0, 0, :]
    dst = e_ref[0, 1, :]
    node_col = lax.broadcasted_iota(jnp.int32, (NODES, E), 0)
    deg = 1.0 + jnp.sum(
        jnp.where(node_col == dst[None, :], 1.0, 0.0), axis=1, keepdims=True)
    dinv = lax.rsqrt(deg)

    lanes = lax.broadcasted_iota(jnp.int32, (E, NODES), 1)
    s_src = jnp.where(lanes == src[:, None], 1.0, 0.0)
    s_dst = jnp.where(lanes == dst[:, None], 1.0, 0.0)
    hs = lax.dot_general(s_src, h, (((1,), (0,)), ((), ())),
                         preferred_element_type=jnp.float32)
    dinv_src = lax.dot_general(s_src, dinv, (((1,), (0,)), ((), ())),
                               preferred_element_type=jnp.float32)
    dinv_dst = lax.dot_general(s_dst, dinv, (((1,), (0,)), ((), ())),
                               preferred_element_type=jnp.float32)
    contrib = hs * (dinv_src * dinv_dst)
    scat = lax.dot_general(s_dst, contrib, (((0,), (0,)), ((), ())),
                           preferred_element_type=jnp.float32)
    out = h * (dinv * dinv) + scat + gb_ref[...]

    wu = wu_ref[...]
    dn = (((1,), (1,)), ((), ()))
    for t in range(V):
        z = lax.dot_general(wu[1], out[t * N:(t + 1) * N, :], dn,
                            preferred_element_type=jnp.float32)
        if t > 0:
            z = z + lax.dot_general(wu[0], out[(t - 1) * N:t * N, :], dn,
                                    preferred_element_type=jnp.float32)
        if t < V - 1:
            z = z + lax.dot_general(wu[2], out[(t + 1) * N:(t + 2) * N, :], dn,
                                    preferred_element_type=jnp.float32)
        z_ref[0, t] = z
        s1_ref[...] += jnp.sum(z, axis=1, keepdims=True)
        s2_ref[...] += jnp.sum(z * z, axis=1, keepdims=True)


def _gcn_conv2(y_flat, edges, scale, shift, gcn_w, gcn_b, wu3):
    return pl.pallas_call(
        _gcn_body,
        grid=(B,),
        in_specs=[
            pl.BlockSpec((1, NODES, C), lambda b: (b, 0, 0)),
            pl.BlockSpec((1, 2, E), lambda b: (b, 0, 0)),
            pl.BlockSpec((1, C), lambda b: (0, 0)),
            pl.BlockSpec((1, C), lambda b: (0, 0)),
            pl.BlockSpec((C, C), lambda b: (0, 0)),
            pl.BlockSpec((1, C), lambda b: (0, 0)),
            pl.BlockSpec((3, C, C), lambda b: (0, 0, 0)),
        ],
        out_specs=[
            pl.BlockSpec((1, V, C, N), lambda b: (b, 0, 0, 0)),
            pl.BlockSpec((C, 1), lambda b: (0, 0)),
            pl.BlockSpec((C, 1), lambda b: (0, 0)),
        ],
        out_shape=[
            jax.ShapeDtypeStruct((B, V, C, N), jnp.float32),
            jax.ShapeDtypeStruct((C, 1), jnp.float32),
            jax.ShapeDtypeStruct((C, 1), jnp.float32),
        ],
    )(y_flat, edges, scale, shift, gcn_w, gcn_b, wu3)


def _affine_body(z_ref, sc_ref, sh_ref, o_ref):
    o_ref[0] = z_ref[0] * sc_ref[...] + sh_ref[...]


def _affine(z, scale, shift):
    return pl.pallas_call(
        _affine_body,
        grid=(B * V,),
        in_specs=[
            pl.BlockSpec((1, C, N), lambda i: (i, 0, 0)),
            pl.BlockSpec((C, 1), lambda i: (0, 0)),
            pl.BlockSpec((C, 1), lambda i: (0, 0)),
        ],
        out_specs=pl.BlockSpec((1, C, N), lambda i: (i, 0, 0)),
        out_shape=jax.ShapeDtypeStruct((B * V, C, N), jnp.float32),
    )(z, scale, shift)


def kernel(x, w_down, bn1_g, bn1_b, gcn_w, gcn_b, w_up, bn2_g, bn2_b,
           batch: Any, span: Any):
    eps = jnp.float32(1e-5)
    dep = jnp.asarray(batch, jnp.float32) / B
    x3 = x.reshape(B * V, C, N)
    w3 = jnp.transpose(w_down[:, :, :, 0, 0], (2, 0, 1)) * dep

    y, s1, s2 = _conv1(x3, w3)
    mean = s1 / CNT
    var = s2 / CNT - mean * mean
    rstd = lax.rsqrt(var + eps)
    scale = bn1_g[None, :] * rstd
    shift = bn1_b[None, :] - mean * scale

    idx = _topk(y, scale, shift).reshape(B, T1, K)
    row = idx // N
    col = idx % N
    offs = jnp.arange(T1, dtype=jnp.int32)
    row_g = (row + offs[None, :, None] * N).reshape(B, T1 * K)
    col_g = (col + (offs[None, :, None] + jnp.asarray(span, jnp.int32)) * N
             ).reshape(B, T1 * K)
    src = jnp.concatenate([row_g, col_g], axis=1)
    dst = jnp.concatenate([col_g, row_g], axis=1)
    edges = jnp.stack([src, dst], axis=1)

    y_flat = y.reshape(B, NODES, C)
    wu3 = jnp.transpose(w_up[:, :, :, 0, 0], (2, 0, 1))
    z, t1, t2 = _gcn_conv2(y_flat, edges, scale, shift, gcn_w, gcn_b[None, :], wu3)

    mean2 = t1 / CNT
    var2 = t2 / CNT - mean2 * mean2
    rstd2 = lax.rsqrt(var2 + eps)
    scale2 = bn2_g[:, None] * rstd2
    shift2 = bn2_b[:, None] - mean2 * scale2

    out = _affine(z.reshape(B * V, C, N), scale2, shift2)
    return out.reshape(B * V, C, 14, 14)
